# baseline (speedup 1.0000x reference)
import functools

import jax
import jax.numpy as jnp
from jax import lax
from jax.experimental import pallas as pl
from jax.experimental.pallas import tpu as pltpu
from jax.experimental.pallas import tpu_sc as plsc

_S = 512
_B = 8
_N = 16384
_F = 64
_W = 128
_C = _N // _W


def _fps_body(x_ref, y_ref, z_ref, idx_ref, sx_ref, sy_ref, sz_ref, dist_ref):
    dist_ref[...] = jnp.full((_C, _B, _W), 1e10, jnp.float32)

    lanes0f = lax.broadcasted_iota(jnp.int32, (_B, _W), 1).astype(jnp.float32)
    lanes_s = lax.broadcasted_iota(jnp.int32, (_B, _S), 1)
    row_off = lax.broadcasted_iota(jnp.int32, (_B, 1), 0) * _N

    cx0 = x_ref[0, :, 0:1]
    cy0 = y_ref[0, :, 0:1]
    cz0 = z_ref[0, :, 0:1]

    def body(i, carry):
        far, cx, cy, cz = carry

        sel = lanes_s == i
        idx_ref[...] = jnp.where(sel, far + row_off, idx_ref[...])
        sx_ref[...] = jnp.where(sel, cx, sx_ref[...])
        sy_ref[...] = jnp.where(sel, cy, sy_ref[...])
        sz_ref[...] = jnp.where(sel, cz, sz_ref[...])

        def chunk(t, acc):
            def one(c, a, lanes):
                vacc, iacc, cxa, cya, cza = a
                xv = x_ref[c]
                yv = y_ref[c]
                zv = z_ref[c]
                dx = xv - cx
                dy = yv - cy
                dz = zv - cz
                d = dx * dx + dy * dy + dz * dz
                dmin = jnp.minimum(dist_ref[c], d)
                dist_ref[c] = dmin
                upd = dmin > vacc
                vacc = jnp.where(upd, dmin, vacc)
                iacc = jnp.where(upd, lanes, iacc)
                cxa = jnp.where(upd, xv, cxa)
                cya = jnp.where(upd, yv, cya)
                cza = jnp.where(upd, zv, cza)
                return (vacc, iacc, cxa, cya, cza)

            return tuple(
                one(4 * t + j, acc[j], lanes0f + (4 * t + j) * float(_W))
                for j in range(4)
            )

        zero_f = jnp.zeros((_B, _W), jnp.float32)
        init = (zero_f - 1.0, zero_f, zero_f, zero_f, zero_f)
        accs = lax.fori_loop(0, _C // 4, chunk, (init,) * 4, unroll=32)

        def combine(a, b):
            v1, i1, x1, y1, z1 = a
            v2, i2, x2, y2, z2 = b
            take2 = (v2 > v1) | ((v2 == v1) & (i2 < i1))
            return (
                jnp.where(take2, v2, v1),
                jnp.where(take2, i2, i1),
                jnp.where(take2, x2, x1),
                jnp.where(take2, y2, y1),
                jnp.where(take2, z2, z1),
            )

        vacc, iacc, cxa, cya, cza = combine(
            combine(accs[0], accs[1]), combine(accs[2], accs[3])
        )

        m = jnp.max(vacc, axis=1, keepdims=True)
        far2f = jnp.min(jnp.where(vacc == m, iacc, jnp.inf), axis=1, keepdims=True)
        wsel = iacc == far2f
        cx2 = jnp.sum(jnp.where(wsel, cxa, 0.0), axis=1, keepdims=True)
        cy2 = jnp.sum(jnp.where(wsel, cya, 0.0), axis=1, keepdims=True)
        cz2 = jnp.sum(jnp.where(wsel, cza, 0.0), axis=1, keepdims=True)
        return (far2f.astype(jnp.int32), cx2, cy2, cz2)

    lax.fori_loop(
        0, _S, body, (jnp.zeros((_B, 1), jnp.int32), cx0, cy0, cz0)
    )


def _run_fps(x, y, z):
    return pl.pallas_call(
        _fps_body,
        out_shape=[
            jax.ShapeDtypeStruct((_B, _S), jnp.int32),
            jax.ShapeDtypeStruct((_B, _S), jnp.float32),
            jax.ShapeDtypeStruct((_B, _S), jnp.float32),
            jax.ShapeDtypeStruct((_B, _S), jnp.float32),
        ],
        scratch_shapes=[pltpu.VMEM((_C, _B, _W), jnp.float32)],
    )(x, y, z)


_NC = 2
_NS = 16
_NW = _NC * _NS
_ROWS_PER_W = (_B * _S) // _NW


def _sc_gather(table_hbm, idx_hbm, out_hbm, idx_v, rows_v, sem):
    wid = lax.axis_index("s") * _NC + lax.axis_index("c")
    base = wid * _ROWS_PER_W
    pltpu.sync_copy(idx_hbm.at[pl.ds(base, _ROWS_PER_W)], idx_v)
    pltpu.async_copy(table_hbm.at[idx_v], rows_v, sem).wait()
    pltpu.sync_copy(rows_v, out_hbm.at[pl.ds(base, _ROWS_PER_W)])


@functools.lru_cache(maxsize=None)
def _gather_pairs_kernel():
    return pl.kernel(
        _sc_gather,
        out_type=jax.ShapeDtypeStruct((_B * _S, 2 * _F), jnp.float32),
        mesh=plsc.VectorSubcoreMesh(
            core_axis_name="c", subcore_axis_name="s", num_cores=_NC, num_subcores=_NS
        ),
        scratch_types=[
            pltpu.VMEM((_ROWS_PER_W,), jnp.int32),
            pltpu.VMEM((_ROWS_PER_W, 2 * _F), jnp.float32),
            pltpu.SemaphoreType.DMA,
        ],
    )


def _half_select_body(pairs_ref, odd_ref, out_ref):
    odd = odd_ref[...] == 1
    out_ref[...] = jnp.where(odd, pairs_ref[:, _F:], pairs_ref[:, :_F])


def _half_select(pairs, odd):
    return pl.pallas_call(
        _half_select_body,
        out_shape=jax.ShapeDtypeStruct((_B * _S, _F), jnp.float32),
    )(pairs, odd)


@jax.jit
def kernel(xyz, f):
    planes = xyz.reshape(_B, _C, _W, 3).transpose(3, 1, 0, 2)
    x, y, z = planes[0], planes[1], planes[2]
    flat_idx, sx, sy, sz = _run_fps(x, y, z)
    xyz_sampled = jnp.stack([sx, sy, sz], axis=-1)
    table = f.reshape(_B * _N // 2, 2 * _F)
    flat = flat_idx.reshape(_B * _S)
    pairs = _gather_pairs_kernel()(table, flat >> 1)
    f_sampled = _half_select(pairs, (flat & 1).reshape(_B * _S, 1)).reshape(
        _B, _S, _F
    )
    return (xyz_sampled, f_sampled)


# speedup vs baseline: 1.0191x; 1.0191x over previous
import functools

import jax
import jax.numpy as jnp
from jax import lax
from jax.experimental import pallas as pl
from jax.experimental.pallas import tpu as pltpu
from jax.experimental.pallas import tpu_sc as plsc

_S = 512
_B = 8
_N = 16384
_F = 64
_W = 128
_C = _N // _W


def _fps_body(x_ref, y_ref, z_ref, idx_ref, sx_ref, sy_ref, sz_ref, dist_ref):
    dist_ref[...] = jnp.full((_C, _B, _W), 1e10, jnp.float32)

    lanes0f = lax.broadcasted_iota(jnp.int32, (_B, _W), 1).astype(jnp.float32)
    lanes_s = lax.broadcasted_iota(jnp.int32, (_B, _S), 1)
    row_off = lax.broadcasted_iota(jnp.int32, (_B, 1), 0) * _N

    cx0 = x_ref[0, :, 0:1]
    cy0 = y_ref[0, :, 0:1]
    cz0 = z_ref[0, :, 0:1]

    def body(i, carry):
        far, cx, cy, cz = carry

        sel = lanes_s == i
        idx_ref[...] = jnp.where(sel, far + row_off, idx_ref[...])
        sx_ref[...] = jnp.where(sel, cx, sx_ref[...])
        sy_ref[...] = jnp.where(sel, cy, sy_ref[...])
        sz_ref[...] = jnp.where(sel, cz, sz_ref[...])

        def chunk(t, acc):
            def one(c, a, lanes):
                vacc, iacc, cxa, cya, cza = a
                xv = x_ref[c]
                yv = y_ref[c]
                zv = z_ref[c]
                dx = xv - cx
                dy = yv - cy
                dz = zv - cz
                d = dx * dx + dy * dy + dz * dz
                dmin = jnp.minimum(dist_ref[c], d)
                dist_ref[c] = dmin
                upd = dmin > vacc
                vacc = jnp.where(upd, dmin, vacc)
                iacc = jnp.where(upd, lanes, iacc)
                cxa = jnp.where(upd, xv, cxa)
                cya = jnp.where(upd, yv, cya)
                cza = jnp.where(upd, zv, cza)
                return (vacc, iacc, cxa, cya, cza)

            return tuple(
                one(4 * t + j, acc[j], lanes0f + (4 * t + j) * float(_W))
                for j in range(4)
            )

        zero_f = jnp.zeros((_B, _W), jnp.float32)
        init = (zero_f - 1.0, zero_f, zero_f, zero_f, zero_f)
        accs = lax.fori_loop(0, _C // 4, chunk, (init,) * 4, unroll=32)

        def combine(a, b):
            v1, i1, x1, y1, z1 = a
            v2, i2, x2, y2, z2 = b
            take2 = (v2 > v1) | ((v2 == v1) & (i2 < i1))
            return (
                jnp.where(take2, v2, v1),
                jnp.where(take2, i2, i1),
                jnp.where(take2, x2, x1),
                jnp.where(take2, y2, y1),
                jnp.where(take2, z2, z1),
            )

        vacc, iacc, cxa, cya, cza = combine(
            combine(accs[0], accs[1]), combine(accs[2], accs[3])
        )

        m = jnp.max(vacc, axis=1, keepdims=True)
        far2f = jnp.min(jnp.where(vacc == m, iacc, jnp.inf), axis=1, keepdims=True)
        wsel = iacc == far2f
        cx2 = jnp.sum(jnp.where(wsel, cxa, 0.0), axis=1, keepdims=True)
        cy2 = jnp.sum(jnp.where(wsel, cya, 0.0), axis=1, keepdims=True)
        cz2 = jnp.sum(jnp.where(wsel, cza, 0.0), axis=1, keepdims=True)
        return (far2f.astype(jnp.int32), cx2, cy2, cz2)

    lax.fori_loop(
        0, _S, body, (jnp.zeros((_B, 1), jnp.int32), cx0, cy0, cz0)
    )


def _run_fps(x, y, z):
    return pl.pallas_call(
        _fps_body,
        out_shape=[
            jax.ShapeDtypeStruct((_B, _S), jnp.int32),
            jax.ShapeDtypeStruct((_B, _S), jnp.float32),
            jax.ShapeDtypeStruct((_B, _S), jnp.float32),
            jax.ShapeDtypeStruct((_B, _S), jnp.float32),
        ],
        scratch_shapes=[pltpu.VMEM((_C, _B, _W), jnp.float32)],
    )(x, y, z)


_NC = 2
_NS = 16
_NW = _NC * _NS
_ROWS_PER_W = (_B * _S) // _NW


def _sc_gather(table_hbm, idx_hbm, out_hbm, idx_v, rows_v, sem):
    wid = lax.axis_index("s") * _NC + lax.axis_index("c")
    base = wid * _ROWS_PER_W
    pltpu.sync_copy(idx_hbm.at[pl.ds(base, _ROWS_PER_W)], idx_v)
    pltpu.async_copy(table_hbm.at[idx_v], rows_v, sem).wait()
    pltpu.sync_copy(rows_v, out_hbm.at[pl.ds(base, _ROWS_PER_W)])


@functools.lru_cache(maxsize=None)
def _gather_rows_kernel():
    return pl.kernel(
        _sc_gather,
        out_type=jax.ShapeDtypeStruct((_B * _S, _F), jnp.float32),
        mesh=plsc.VectorSubcoreMesh(
            core_axis_name="c", subcore_axis_name="s", num_cores=_NC, num_subcores=_NS
        ),
        scratch_types=[
            pltpu.VMEM((_ROWS_PER_W,), jnp.int32),
            pltpu.VMEM((_ROWS_PER_W, _F), jnp.float32),
            pltpu.SemaphoreType.DMA,
        ],
        compiler_params=pltpu.CompilerParams(use_tc_tiling_on_sc=False),
    )


@jax.jit
def kernel(xyz, f):
    xcw = xyz.reshape(_B, _C, _W, 3)
    x = xcw[:, :, :, 0].transpose(1, 0, 2)
    y = xcw[:, :, :, 1].transpose(1, 0, 2)
    z = xcw[:, :, :, 2].transpose(1, 0, 2)
    flat_idx, sx, sy, sz = _run_fps(x, y, z)
    xyz_sampled = jnp.stack([sx, sy, sz], axis=-1)
    table = f.reshape(_B * _N, _F)
    f_sampled = _gather_rows_kernel()(table, flat_idx.reshape(_B * _S)).reshape(
        _B, _S, _F
    )
    return (xyz_sampled, f_sampled)
